# baseline (speedup 1.0000x reference)
import functools

import numpy as np
import jax
import jax.numpy as jnp
from jax.experimental import pallas as pl
from jax.experimental.pallas import tpu as pltpu

IMAGE_SIZE = 224
MEAN = np.array([131.0912, 103.8827, 91.4953], dtype=np.float32)
STD = np.array([1.0, 1.0, 1.0], dtype=np.float32)
_MEAN_OVER_STD = (MEAN / STD).astype(np.float32)
_INV_STD = (1.0 / STD).astype(np.float32)
_STD_IS_ONE = bool(np.all(STD == 1.0))


def _channel_column(rows, C, values):
    c = jax.lax.broadcasted_iota(jnp.int32, (rows, 1), 0) // IMAGE_SIZE
    col = jnp.full((rows, 1), float(values[C - 1]), jnp.float32)
    for j in range(C - 1):
        col = jnp.where(c == j, float(values[j]), col)
    return col


def _norm_body(x_ref, o_ref, *, C):
    x = x_ref[...].astype(jnp.float32)
    rows = x_ref.shape[1]
    mean = _channel_column(rows, C, _MEAN_OVER_STD)
    if _STD_IS_ONE:
        o_ref[...] = x - mean
    else:
        inv = _channel_column(rows, C, _INV_STD)
        o_ref[...] = x * inv - mean


def _pick_batch_block(B, C, itemsize):
    per_image = C * IMAGE_SIZE * IMAGE_SIZE * max(itemsize, 4)
    best = 1
    for nb in range(1, B + 1):
        if B % nb:
            continue
        if nb * per_image > (3 << 20):
            continue
        if B // nb < 8 and nb > 1:
            continue
        best = nb
    return best


def kernel(image):
    B, C, H, W = image.shape
    if H != IMAGE_SIZE or W != IMAGE_SIZE:
        raise ValueError(f"expected {IMAGE_SIZE}x{IMAGE_SIZE} input, got {H}x{W}")
    rows = C * IMAGE_SIZE
    x = image.reshape(B, rows, IMAGE_SIZE)
    nb = _pick_batch_block(B, C, np.dtype(image.dtype).itemsize)
    out = pl.pallas_call(
        functools.partial(_norm_body, C=C),
        out_shape=jax.ShapeDtypeStruct((B, rows, IMAGE_SIZE), jnp.float32),
        grid=(B // nb,),
        in_specs=[pl.BlockSpec((nb, rows, IMAGE_SIZE), lambda i: (i, 0, 0))],
        out_specs=pl.BlockSpec((nb, rows, IMAGE_SIZE), lambda i: (i, 0, 0)),
        compiler_params=pltpu.CompilerParams(
            dimension_semantics=("parallel",)),
    )(x)
    return out.reshape(B, C, IMAGE_SIZE, IMAGE_SIZE)


# speedup vs baseline: 1.1370x; 1.1370x over previous
import functools

import numpy as np
import jax
import jax.numpy as jnp
from jax.experimental import pallas as pl
from jax.experimental.pallas import tpu as pltpu

IMAGE_SIZE = 224
MEAN = np.array([131.0912, 103.8827, 91.4953], dtype=np.float32)
STD = np.array([1.0, 1.0, 1.0], dtype=np.float32)
_MEAN_OVER_STD = (MEAN / STD).astype(np.float32)
_INV_STD = (1.0 / STD).astype(np.float32)
_STD_IS_ONE = bool(np.all(STD == 1.0))


def _channel_column(rows, C, values):
    c = jax.lax.broadcasted_iota(jnp.int32, (rows, 1), 0) // IMAGE_SIZE
    col = jnp.full((rows, 1), float(values[C - 1]), jnp.float32)
    for j in range(C - 1):
        col = jnp.where(c == j, float(values[j]), col)
    return col


def _norm_body(x_ref, o_ref, *, C):
    x = x_ref[...].astype(jnp.float32)
    rows = x_ref.shape[1]
    mean = _channel_column(rows, C, _MEAN_OVER_STD)
    if _STD_IS_ONE:
        o_ref[...] = x - mean
    else:
        inv = _channel_column(rows, C, _INV_STD)
        o_ref[...] = x * inv - mean


def _pick_batch_block(B, C, itemsize):
    per_image = C * IMAGE_SIZE * IMAGE_SIZE * max(itemsize, 4)
    best = 1
    for nb in range(1, B + 1):
        if B % nb:
            continue
        if nb * per_image > (10 << 20):
            continue
        if B // nb < 4 and nb > 1:
            continue
        best = nb
    return best


def kernel(image):
    B, C, H, W = image.shape
    if H != IMAGE_SIZE or W != IMAGE_SIZE:
        raise ValueError(f"expected {IMAGE_SIZE}x{IMAGE_SIZE} input, got {H}x{W}")
    rows = C * IMAGE_SIZE
    x = image.reshape(B, rows, IMAGE_SIZE)
    nb = _pick_batch_block(B, C, np.dtype(image.dtype).itemsize)
    out = pl.pallas_call(
        functools.partial(_norm_body, C=C),
        out_shape=jax.ShapeDtypeStruct((B, rows, IMAGE_SIZE), jnp.float32),
        grid=(B // nb,),
        in_specs=[pl.BlockSpec((nb, rows, IMAGE_SIZE), lambda i: (i, 0, 0))],
        out_specs=pl.BlockSpec((nb, rows, IMAGE_SIZE), lambda i: (i, 0, 0)),
        compiler_params=pltpu.CompilerParams(
            dimension_semantics=("parallel",)),
    )(x)
    return out.reshape(B, C, IMAGE_SIZE, IMAGE_SIZE)
